# baseline (speedup 1.0000x reference)
import functools

import jax
import jax.numpy as jnp
from jax import lax
from jax.experimental import pallas as pl
from jax.experimental.pallas import tpu as pltpu
from jax.experimental.pallas import tpu_sc as plsc

V = 1000
V_PAD = 1024
BT = 20480
NC, NS, L = 2, 16, 16
NW = NC * NS
B_PER_W = BT // NW
PC = 32
N_CHUNKS = B_PER_W // PC
VT = V // 8
PT = BT // 128


def _lse_body(table_ref, lse_ref):
    x = table_ref[...]
    m = jnp.max(x, axis=1)
    s = jnp.sum(jnp.exp(x - m[:, None]), axis=1)
    lse_ref[pl.ds(0, V)] = m + jnp.log(s)
    lse_ref[pl.ds(V, V_PAD - V)] = jnp.zeros((V_PAD - V,), jnp.float32)


def _row_lse(table):
    return pl.pallas_call(
        _lse_body,
        out_shape=jax.ShapeDtypeStruct((V_PAD,), jnp.float32),
    )(table)


_MESH = plsc.VectorSubcoreMesh(core_axis_name="c", subcore_axis_name="s")


@functools.partial(
    pl.kernel,
    mesh=_MESH,
    compiler_params=pltpu.CompilerParams(
        use_tc_tiling_on_sc=False, needs_layout_passes=False
    ),
    out_type=[
        jax.ShapeDtypeStruct((VT, PT, 8, 128), jnp.float32),
        jax.ShapeDtypeStruct((NW, L), jnp.float32),
    ],
    scratch_types=[
        pltpu.VMEM((B_PER_W,), jnp.int32),
        pltpu.VMEM((B_PER_W,), jnp.int32),
        pltpu.VMEM((V_PAD,), jnp.float32),
        pltpu.VMEM((PC, V), jnp.float32),
        pltpu.VMEM((PC, V), jnp.float32),
        pltpu.VMEM((VT, 8, PC), jnp.float32),
        pltpu.VMEM((VT, 8, PC), jnp.float32),
        pltpu.VMEM((L,), jnp.float32),
        pltpu.SemaphoreType.DMA,
        pltpu.SemaphoreType.DMA,
        pltpu.SemaphoreType.DMA,
        pltpu.SemaphoreType.DMA,
    ],
)
def _sc_gather(table_hbm, idx_hbm, tgt_hbm, lse_hbm, out_hbm, loss_hbm,
               idx_v, tgt_v, lse_v, rows0_v, rows1_v, tr0_v, tr1_v, acc_v,
               semg0, semg1, semw0, semw1):
    wid = lax.axis_index("s") * NC + lax.axis_index("c")
    base = wid * B_PER_W
    pltpu.sync_copy(idx_hbm.at[pl.ds(base, B_PER_W)], idx_v)
    pltpu.sync_copy(tgt_hbm.at[pl.ds(base, B_PER_W)], tgt_v)
    pltpu.sync_copy(lse_hbm, lse_v)
    lane = lax.iota(jnp.int32, L)
    acc = jnp.zeros((L,), jnp.float32)
    rows = (rows0_v, rows1_v)
    trs = (tr0_v, tr1_v)
    semg = (semg0, semg1)
    semw = (semw0, semw1)
    pend_g = pltpu.async_copy(
        table_hbm.at[idx_v.at[pl.ds(0, PC)]], rows0_v, semg0
    )
    pend_w = [None, None]

    for c in range(N_CHUNKS):
        buf = rows[c % 2]
        tr = trs[c % 2]
        pend_g.wait()
        if c + 1 < N_CHUNKS:
            pend_g = pltpu.async_copy(
                table_hbm.at[idx_v.at[pl.ds((c + 1) * PC, PC)]],
                rows[(c + 1) % 2], semg[(c + 1) % 2],
            )
        if pend_w[c % 2] is not None:
            pend_w[c % 2].wait()

        def _tr_body(i, _, buf=buf, tr=tr):
            for s in range(8):
                vvec = jnp.zeros((L,), jnp.int32) + (i * 8 + s)
                for k in range(PC // L):
                    vals = plsc.load_gather(buf, [lane + (k * L), vvec])
                    tr[i, s, pl.ds(k * L, L)] = vals
            return 0

        lax.fori_loop(0, VT, _tr_body, 0, unroll=False)

        p0 = base + c * PC
        jtile = p0 // 128
        lo = p0 % 128
        pend_w[c % 2] = pltpu.async_copy(
            tr, out_hbm.at[:, jtile, :, pl.ds(lo, PC)], semw[c % 2]
        )

        for j in range(PC // L):
            p = c * PC + j * L
            ids = idx_v[pl.ds(p, L)]
            tgs = tgt_v[pl.ds(p, L)]
            vals = plsc.load_gather(buf, [lane + (j * L), tgs])
            lses = plsc.load_gather(lse_v, [ids])
            acc = acc + (lses - vals)

    for k in range(2):
        if pend_w[k] is not None:
            pend_w[k].wait()
    acc_v[...] = acc * (1.0 / BT)
    pltpu.sync_copy(acc_v, loss_hbm.at[wid])


def kernel(idx, targets, table):
    idx_f = idx.reshape(-1).astype(jnp.int32)
    tgt_f = targets.reshape(-1).astype(jnp.int32)
    lse = _row_lse(table)
    out4, loss_part = _sc_gather(table, idx_f, tgt_f, lse)
    logits2 = out4.transpose(1, 3, 0, 2).reshape(BT, V)
    loss = jnp.sum(loss_part)
    return (logits2, loss)


# speedup vs baseline: 1.1398x; 1.1398x over previous
import functools

import jax
import jax.numpy as jnp
from jax import lax
from jax.experimental import pallas as pl
from jax.experimental.pallas import tpu as pltpu
from jax.experimental.pallas import tpu_sc as plsc

V = 1000
V_PAD = 1024
BT = 20480
NC, NS, L = 2, 16, 16
NW = NC * NS
B_PER_W = BT // NW
VT = V // 8
PT = BT // 128
NT = 4


def _prep_body(table_ref, lse_ref, tt_ref):
    x = table_ref[...]
    m = jnp.max(x, axis=1)
    s = jnp.sum(jnp.exp(x - m[:, None]), axis=1)
    lse_ref[pl.ds(0, V)] = m + jnp.log(s)
    lse_ref[pl.ds(V, V_PAD - V)] = jnp.zeros((V_PAD - V,), jnp.float32)
    tt_ref[...] = x.T


def _prep(table):
    return pl.pallas_call(
        _prep_body,
        out_shape=[
            jax.ShapeDtypeStruct((V_PAD,), jnp.float32),
            jax.ShapeDtypeStruct((V, V), jnp.float32),
        ],
    )(table)


_MESH = plsc.VectorSubcoreMesh(core_axis_name="c", subcore_axis_name="s")


@functools.partial(
    pl.kernel,
    mesh=_MESH,
    compiler_params=pltpu.CompilerParams(
        use_tc_tiling_on_sc=False, needs_layout_passes=False
    ),
    out_type=[
        jax.ShapeDtypeStruct((VT, PT, 8, 128), jnp.float32),
        jax.ShapeDtypeStruct((NW, L), jnp.float32),
    ],
    scratch_types=[
        pltpu.VMEM((BT,), jnp.int32),
        pltpu.VMEM((BT,), jnp.int32),
        pltpu.VMEM((V_PAD,), jnp.float32),
        pltpu.VMEM((NT * 8, V), jnp.float32),
        pltpu.VMEM((NT, 8, 128), jnp.float32),
        pltpu.VMEM((NT, 8, 128), jnp.float32),
        pltpu.VMEM((L,), jnp.float32),
        pltpu.SemaphoreType.DMA,
        pltpu.SemaphoreType.DMA,
    ],
)
def _sc_fill(tt_hbm, idx_hbm, tgt_hbm, lse_hbm, out_hbm, loss_hbm,
             idx_v, tgt_v, lse_v, slab_v, ob0_v, ob1_v, acc_v, semw0, semw1):
    wid = lax.axis_index("s") * NC + lax.axis_index("c")
    start = (wid * (VT - NT)) // (NW - 1)
    vbase = start * 8
    pltpu.sync_copy(tt_hbm.at[pl.ds(vbase, NT * 8)], slab_v)
    pltpu.sync_copy(idx_hbm, idx_v)
    pltpu.sync_copy(tgt_hbm, tgt_v)
    pltpu.sync_copy(lse_hbm, lse_v)

    def _fill(jtile, ob):
        idxk = [idx_v[pl.ds(jtile * 128 + k * L, L)] for k in range(8)]
        def _tile(ti, _):
            for s in range(8):
                rvec = jnp.zeros((L,), jnp.int32) + (ti * 8 + s)
                for k in range(8):
                    vals = plsc.load_gather(slab_v, [rvec, idxk[k]])
                    ob[ti, s, pl.ds(k * L, L)] = vals
            return 0
        lax.fori_loop(0, NT, _tile, 0, unroll=False)

    def _write(jtile, ob, sem):
        return pltpu.async_copy(ob, out_hbm.at[pl.ds(start, NT), jtile], sem)

    _fill(0, ob0_v)
    w0 = _write(0, ob0_v, semw0)
    _fill(1, ob1_v)
    w1 = _write(1, ob1_v, semw1)

    def _step(jj, _):
        pltpu.make_async_copy(ob0_v, out_hbm.at[pl.ds(start, NT), 0], semw0).wait()
        _fill(2 * jj, ob0_v)
        pltpu.async_copy(ob0_v, out_hbm.at[pl.ds(start, NT), 2 * jj], semw0)
        pltpu.make_async_copy(ob1_v, out_hbm.at[pl.ds(start, NT), 0], semw1).wait()
        _fill(2 * jj + 1, ob1_v)
        pltpu.async_copy(ob1_v, out_hbm.at[pl.ds(start, NT), 2 * jj + 1], semw1)
        return 0

    lax.fori_loop(1, PT // 2, _step, 0, unroll=False)

    lane = lax.iota(jnp.int32, L)
    acc = jnp.zeros((L,), jnp.float32)
    pbase = wid * B_PER_W
    def _lse_part(j, acc):
        ids = idx_v[pl.ds(pbase + j * L, L)]
        return acc + plsc.load_gather(lse_v, [ids])
    acc = lax.fori_loop(0, B_PER_W // L, _lse_part, acc, unroll=False)

    nxt = jnp.where(wid < NW - 1, ((wid + 1) * (VT - NT)) // (NW - 1), VT)
    vlo = vbase
    vhi = nxt * 8
    def _tgt_part(j, acc):
        ids = idx_v[pl.ds(j * L, L)]
        tgs = tgt_v[pl.ds(j * L, L)]
        m = (tgs >= vlo) & (tgs < vhi)
        vloc = jnp.clip(tgs - vlo, 0, NT * 8 - 1)
        vals = plsc.load_gather(slab_v, [vloc, ids])
        return acc - jnp.where(m, vals, jnp.zeros((L,), jnp.float32))
    acc = lax.fori_loop(0, BT // L, _tgt_part, acc, unroll=False)

    pltpu.make_async_copy(ob0_v, out_hbm.at[pl.ds(start, NT), 0], semw0).wait()
    pltpu.make_async_copy(ob1_v, out_hbm.at[pl.ds(start, NT), 0], semw1).wait()
    acc_v[...] = acc * (1.0 / BT)
    pltpu.sync_copy(acc_v, loss_hbm.at[wid])
    del w0, w1


def kernel(idx, targets, table):
    idx_f = idx.reshape(-1).astype(jnp.int32)
    tgt_f = targets.reshape(-1).astype(jnp.int32)
    lse, tt = _prep(table)
    out4, loss_part = _sc_fill(tt, idx_f, tgt_f, lse)
    logits2 = out4.transpose(1, 3, 0, 2).reshape(BT, V)
    loss = jnp.sum(loss_part)
    return (logits2, loss)


# speedup vs baseline: 1.6328x; 1.4326x over previous
import functools

import jax
import jax.numpy as jnp
from jax import lax
from jax.experimental import pallas as pl
from jax.experimental.pallas import tpu as pltpu
from jax.experimental.pallas import tpu_sc as plsc

V = 1000
V_PAD = 1024
BT = 20480
NC, NS, L = 2, 16, 16
NW = NC * NS
B_PER_W = BT // NW
VT = V // 8
PT = BT // 128
NT = 4


def _prep_body(table_ref, lse_ref, tt_ref):
    x = table_ref[...]
    m = jnp.max(x, axis=1)
    s = jnp.sum(jnp.exp(x - m[:, None]), axis=1)
    lse_ref[pl.ds(0, V)] = m + jnp.log(s)
    lse_ref[pl.ds(V, V_PAD - V)] = jnp.zeros((V_PAD - V,), jnp.float32)
    tt_ref[...] = x.T


def _prep(table):
    return pl.pallas_call(
        _prep_body,
        out_shape=[
            jax.ShapeDtypeStruct((V_PAD,), jnp.float32),
            jax.ShapeDtypeStruct((V, V), jnp.float32),
        ],
    )(table)


_MESH = plsc.VectorSubcoreMesh(core_axis_name="c", subcore_axis_name="s")


@functools.partial(
    pl.kernel,
    mesh=_MESH,
    compiler_params=pltpu.CompilerParams(
        use_tc_tiling_on_sc=False, needs_layout_passes=False
    ),
    out_type=[
        jax.ShapeDtypeStruct((VT, PT, 8, 128), jnp.float32),
        jax.ShapeDtypeStruct((NW, L), jnp.float32),
    ],
    scratch_types=[
        pltpu.VMEM((BT,), jnp.int32),
        pltpu.VMEM((BT,), jnp.int32),
        pltpu.VMEM((V_PAD,), jnp.float32),
        pltpu.VMEM((NT * 8, V), jnp.float32),
        pltpu.VMEM((NT, 8, 128), jnp.float32),
        pltpu.VMEM((NT, 8, 128), jnp.float32),
        pltpu.VMEM((L,), jnp.float32),
        pltpu.SemaphoreType.DMA,
        pltpu.SemaphoreType.DMA,
    ],
)
def _sc_fill(tt_hbm, idx_hbm, tgt_hbm, lse_hbm, out_hbm, loss_hbm,
             idx_v, tgt_v, lse_v, slab_v, ob0_v, ob1_v, acc_v, semw0, semw1):
    wid = lax.axis_index("s") * NC + lax.axis_index("c")
    start = (wid * (VT - NT)) // (NW - 1)
    vbase = start * 8
    pltpu.sync_copy(tt_hbm.at[pl.ds(vbase, NT * 8)], slab_v)
    pltpu.sync_copy(idx_hbm, idx_v)
    pltpu.sync_copy(tgt_hbm, tgt_v)
    pltpu.sync_copy(lse_hbm, lse_v)

    def _fill(jtile, ob):
        idxk = [idx_v[pl.ds(jtile * 128 + k * L, L)] for k in range(8)]

        @plsc.parallel_loop(0, NT, 1, unroll=2)
        def _tile(ti):
            for s in range(8):
                rvec = jnp.zeros((L,), jnp.int32) + (ti * 8 + s)
                for k in range(8):
                    vals = plsc.load_gather(slab_v, [rvec, idxk[k]])
                    ob[ti, s, pl.ds(k * L, L)] = vals

    def _write(jtile, ob, sem):
        return pltpu.async_copy(ob, out_hbm.at[pl.ds(start, NT), jtile], sem)

    _fill(0, ob0_v)
    w0 = _write(0, ob0_v, semw0)
    _fill(1, ob1_v)
    w1 = _write(1, ob1_v, semw1)

    def _step(jj, _):
        pltpu.make_async_copy(ob0_v, out_hbm.at[pl.ds(start, NT), 0], semw0).wait()
        _fill(2 * jj, ob0_v)
        pltpu.async_copy(ob0_v, out_hbm.at[pl.ds(start, NT), 2 * jj], semw0)
        pltpu.make_async_copy(ob1_v, out_hbm.at[pl.ds(start, NT), 0], semw1).wait()
        _fill(2 * jj + 1, ob1_v)
        pltpu.async_copy(ob1_v, out_hbm.at[pl.ds(start, NT), 2 * jj + 1], semw1)
        return 0

    lax.fori_loop(1, PT // 2, _step, 0, unroll=False)

    lane = lax.iota(jnp.int32, L)
    acc = jnp.zeros((L,), jnp.float32)
    pbase = wid * B_PER_W
    def _lse_part(j, acc):
        ids = idx_v[pl.ds(pbase + j * L, L)]
        return acc + plsc.load_gather(lse_v, [ids])
    acc = lax.fori_loop(0, B_PER_W // L, _lse_part, acc, unroll=False)

    nxt = jnp.where(wid < NW - 1, ((wid + 1) * (VT - NT)) // (NW - 1), VT)
    vlo = vbase
    vhi = nxt * 8
    def _tgt_part(j, acc):
        ids = idx_v[pl.ds(j * L, L)]
        tgs = tgt_v[pl.ds(j * L, L)]
        m = (tgs >= vlo) & (tgs < vhi)
        vloc = jnp.clip(tgs - vlo, 0, NT * 8 - 1)
        vals = plsc.load_gather(slab_v, [vloc, ids])
        return acc - jnp.where(m, vals, jnp.zeros((L,), jnp.float32))
    acc = lax.fori_loop(0, BT // L, _tgt_part, acc, unroll=False)

    pltpu.make_async_copy(ob0_v, out_hbm.at[pl.ds(start, NT), 0], semw0).wait()
    pltpu.make_async_copy(ob1_v, out_hbm.at[pl.ds(start, NT), 0], semw1).wait()
    acc_v[...] = acc * (1.0 / BT)
    pltpu.sync_copy(acc_v, loss_hbm.at[wid])
    del w0, w1


def kernel(idx, targets, table):
    idx_f = idx.reshape(-1).astype(jnp.int32)
    tgt_f = targets.reshape(-1).astype(jnp.int32)
    lse, tt = _prep(table)
    out4, loss_part = _sc_fill(tt, idx_f, tgt_f, lse)
    logits2 = out4.transpose(1, 3, 0, 2).reshape(BT, V)
    loss = jnp.sum(loss_part)
    return (logits2, loss)


# speedup vs baseline: 2.1939x; 1.3436x over previous
import functools

import jax
import jax.numpy as jnp
from jax import lax
from jax.experimental import pallas as pl
from jax.experimental.pallas import tpu as pltpu
from jax.experimental.pallas import tpu_sc as plsc

V = 1000
V_PAD = 1024
BT = 20480
NC, NS, L = 2, 16, 16
NW = NC * NS
B_PER_W = BT // NW
VT = V // 8
PT = BT // 128
NT = 4


def _prep_body(table_ref, lse_ref, tt_ref):
    x = table_ref[...]
    m = jnp.max(x, axis=1)
    s = jnp.sum(jnp.exp(x - m[:, None]), axis=1)
    lse_ref[pl.ds(0, V)] = m + jnp.log(s)
    lse_ref[pl.ds(V, V_PAD - V)] = jnp.zeros((V_PAD - V,), jnp.float32)
    tt_ref[...] = x.T


def _prep(table):
    return pl.pallas_call(
        _prep_body,
        out_shape=[
            jax.ShapeDtypeStruct((V_PAD,), jnp.float32),
            jax.ShapeDtypeStruct((V, V), jnp.float32),
        ],
    )(table)


_MESH = plsc.VectorSubcoreMesh(core_axis_name="c", subcore_axis_name="s")


@functools.partial(
    pl.kernel,
    mesh=_MESH,
    compiler_params=pltpu.CompilerParams(
        use_tc_tiling_on_sc=False, needs_layout_passes=False
    ),
    out_type=[
        jax.ShapeDtypeStruct((VT, PT, 8, 128), jnp.float32),
        jax.ShapeDtypeStruct((NW, L), jnp.float32),
    ],
    scratch_types=[
        pltpu.VMEM((BT,), jnp.int32),
        pltpu.VMEM((BT,), jnp.int32),
        pltpu.VMEM((V_PAD,), jnp.float32),
        pltpu.VMEM((NT * 8, V), jnp.float32),
        pltpu.VMEM((NT, 8, 128), jnp.float32),
        pltpu.VMEM((NT, 8, 128), jnp.float32),
        pltpu.VMEM((L,), jnp.float32),
        pltpu.SemaphoreType.DMA,
        pltpu.SemaphoreType.DMA,
    ],
)
def _sc_fill(tt_hbm, idx_hbm, tgt_hbm, lse_hbm, out_hbm, loss_hbm,
             idx_v, tgt_v, lse_v, slab_v, ob0_v, ob1_v, acc_v, semw0, semw1):
    wid = lax.axis_index("s") * NC + lax.axis_index("c")
    start = (wid * (VT - NT)) // (NW - 1)
    vbase = start * 8
    pltpu.sync_copy(tt_hbm.at[pl.ds(vbase, NT * 8)], slab_v)
    pltpu.sync_copy(idx_hbm, idx_v)
    pltpu.sync_copy(tgt_hbm, tgt_v)
    pltpu.sync_copy(lse_hbm, lse_v)

    def _fill(jtile, ob):
        idxk = [idx_v[pl.ds(jtile * 128 + k * L, L)] for k in range(8)]

        @plsc.parallel_loop(0, NT, 1, unroll=4)
        def _tile(ti):
            for s in range(8):
                rvec = jnp.zeros((L,), jnp.int32) + (ti * 8 + s)
                for k in range(8):
                    vals = plsc.load_gather(slab_v, [rvec, idxk[k]])
                    ob[ti, s, pl.ds(k * L, L)] = vals

    def _write(jtile, ob, sem):
        return pltpu.async_copy(ob, out_hbm.at[pl.ds(start, NT), jtile], sem)

    _fill(0, ob0_v)
    w0 = _write(0, ob0_v, semw0)
    _fill(1, ob1_v)
    w1 = _write(1, ob1_v, semw1)

    def _step(jj, _):
        pltpu.make_async_copy(ob0_v, out_hbm.at[pl.ds(start, NT), 0], semw0).wait()
        _fill(2 * jj, ob0_v)
        pltpu.async_copy(ob0_v, out_hbm.at[pl.ds(start, NT), 2 * jj], semw0)
        pltpu.make_async_copy(ob1_v, out_hbm.at[pl.ds(start, NT), 0], semw1).wait()
        _fill(2 * jj + 1, ob1_v)
        pltpu.async_copy(ob1_v, out_hbm.at[pl.ds(start, NT), 2 * jj + 1], semw1)
        return 0

    lax.fori_loop(1, PT // 2, _step, 0, unroll=False)

    lane = lax.iota(jnp.int32, L)
    acc = jnp.zeros((L,), jnp.float32)
    pbase = wid * B_PER_W
    def _lse_part(j, acc):
        ids = idx_v[pl.ds(pbase + j * L, L)]
        return acc + plsc.load_gather(lse_v, [ids])
    acc = lax.fori_loop(0, B_PER_W // L, _lse_part, acc, unroll=False)

    nxt = jnp.where(wid < NW - 1, ((wid + 1) * (VT - NT)) // (NW - 1), VT)
    vlo = vbase
    vhi = nxt * 8
    def _tgt_part(j, acc):
        ids = idx_v[pl.ds(j * L, L)]
        tgs = tgt_v[pl.ds(j * L, L)]
        m = (tgs >= vlo) & (tgs < vhi)
        vloc = jnp.clip(tgs - vlo, 0, NT * 8 - 1)
        vals = plsc.load_gather(slab_v, [vloc, ids])
        return acc - jnp.where(m, vals, jnp.zeros((L,), jnp.float32))
    acc = lax.fori_loop(0, BT // L, _tgt_part, acc, unroll=False)

    pltpu.make_async_copy(ob0_v, out_hbm.at[pl.ds(start, NT), 0], semw0).wait()
    pltpu.make_async_copy(ob1_v, out_hbm.at[pl.ds(start, NT), 0], semw1).wait()
    acc_v[...] = acc * (1.0 / BT)
    pltpu.sync_copy(acc_v, loss_hbm.at[wid])
    del w0, w1


def kernel(idx, targets, table):
    idx_f = idx.reshape(-1).astype(jnp.int32)
    tgt_f = targets.reshape(-1).astype(jnp.int32)
    lse, tt = _prep(table)
    out4, loss_part = _sc_fill(tt, idx_f, tgt_f, lse)
    logits2 = out4.transpose(1, 3, 0, 2).reshape(BT, V)
    loss = jnp.sum(loss_part)
    return (logits2, loss)
